# Initial kernel scaffold; baseline (speedup 1.0000x reference)
import jax
import jax.numpy as jnp
from jax.experimental import pallas as pl


def kernel(context_unixReviewTime, context_itempos, behaviour_itemId, behaviour_itemCat, candidate_itemId, candidate_itemCat, emb_unixReviewTime, emb_itempos, emb_itemId, emb_itemCat, W1, b1, g1, be1, alpha1, W2, b2, g2, be2, alpha2, W3, b3):
    raise NotImplementedError("write your pallas kernel here")


# speedup vs baseline: 7.2995x; 7.2995x over previous
import functools

import jax
import jax.numpy as jnp
from jax import lax
from jax.experimental import pallas as pl
from jax.experimental.pallas import tpu as pltpu
from jax.experimental.pallas import tpu_sc as plsc

NC, NS, LANES = 2, 16, 16
NW = NC * NS
BN_EPS = 1e-5
CL = 1


def _sc_pooled_gather(idx_urt, idx_pos, idx_bid, idx_bcat, cand_id, cand_cat,
                      t_urt, t_pos, t_id, t_cat):
    L, B = idx_urt.shape
    D = t_id.shape[1]
    bpw = B // NW
    bsc = B // NC
    nch = L // CL
    npair = nch // 2
    mesh = plsc.VectorSubcoreMesh(core_axis_name="c", subcore_axis_name="s",
                                  num_cores=NC, num_subcores=NS)

    @functools.partial(
        pl.kernel,
        out_type=jax.ShapeDtypeStruct((6, B, D), jnp.float32),
        mesh=mesh,
        scratch_types=[
            pltpu.VMEM((L, bpw), jnp.int32),
            pltpu.VMEM((bpw, D), jnp.float32),
            pltpu.VMEM((bpw, D), jnp.float32),
            pltpu.VMEM((6, bpw), jnp.int32),
            pltpu.VMEM((bpw, D), jnp.float32),
            pltpu.VMEM_SHARED((6 * (B // NC), D), jnp.float32),
            pltpu.SemaphoreType.DMA,
            pltpu.SemaphoreType.DMA,
        ],
        compiler_params=pltpu.CompilerParams(use_tc_tiling_on_sc=False),
    )
    def k(urt_h, pos_h, bid_h, bcat_h, cid_h, ccat_h,
          turt_h, tpos_h, tid_h, tcat_h, out_h,
          idxv, stga, stgb, dstv, zbuf, acc, sema, semb):
        c = lax.axis_index("c")
        s = lax.axis_index("s")
        lb = s * bpw
        gb = c * bsc + lb
        lanes = lax.iota(jnp.int32, LANES)
        zeros16 = jnp.zeros((LANES,), jnp.float32)

        for f in range(6):
            for g in range(bpw // LANES):
                dstv[f, pl.ds(g * LANES, LANES)] = (
                    f * bsc + lb + g * LANES + lanes)
        for r in range(bpw):
            for g in range(D // LANES):
                zbuf[r, pl.ds(g * LANES, LANES)] = zeros16
        for f in range(6):
            pltpu.sync_copy(zbuf, acc.at[pl.ds(f * bsc + lb, bpw), :])

        fields = ((urt_h, turt_h, 0), (pos_h, tpos_h, 1),
                  (bid_h, tid_h, 2), (bcat_h, tcat_h, 3))
        for idx_h, tab_h, f in fields:
            pltpu.sync_copy(idx_h.at[:, pl.ds(gb, bpw)], idxv)
            dsl = dstv.at[f]

            def fire(ci, stg, sem, tab_h=tab_h):
                pltpu.async_copy(tab_h.at[idxv.at[ci]], stg, sem)

            def wait(stg, sem, tab_h=tab_h):
                pltpu.make_async_copy(
                    tab_h.at[idxv.at[0]], stg, sem).wait()

            fire(0, stga, sema)

            def body(p, carry, fire=fire, wait=wait, dsl=dsl):
                wait(stga, sema)
                fire(2 * p + 1, stgb, semb)
                pltpu.sync_copy(stga, acc.at[dsl], add=True)
                @pl.when(p < npair - 1)
                def _():
                    fire(2 * p + 2, stga, sema)
                wait(stgb, semb)
                pltpu.sync_copy(stgb, acc.at[dsl], add=True)
                return carry

            lax.fori_loop(0, npair, body, 0)

        cands = ((cid_h, tid_h, 4), (ccat_h, tcat_h, 5))
        for idx_h, tab_h, f in cands:
            pltpu.sync_copy(idx_h.at[0, pl.ds(gb, bpw)], idxv.at[0])
            pltpu.async_copy(tab_h.at[idxv.at[0]], stga, sema).wait()
            pltpu.sync_copy(stga, acc.at[dstv.at[f]], add=True)

        for f in range(6):
            pltpu.sync_copy(acc.at[pl.ds(f * bsc + lb, bpw), :],
                            out_h.at[f, pl.ds(gb, bpw), :])

    return k(idx_urt, idx_pos, idx_bid, idx_bcat,
             cand_id.reshape(1, B), cand_cat.reshape(1, B),
             t_urt, t_pos, t_id, t_cat)


def _mlp_body(x_ref, w1_ref, b1_ref, g1_ref, be1_ref, a1_ref,
              w2_ref, b2_ref, g2_ref, be2_ref, a2_ref,
              w3_ref, b3_ref, o_ref, h1s, h2s):
    B = x_ref.shape[0]
    TB = 512
    T = B // TB

    def mm(a, b):
        return lax.dot_general(a, b, (((1,), (0,)), ((), ())),
                               preferred_element_type=jnp.float32,
                               precision=lax.Precision.HIGHEST)

    def bn(h, g_ref, be_ref):
        return (h * (g_ref[0] / jnp.sqrt(1.0 + BN_EPS))[None, :]
                + be_ref[0][None, :])

    def layer(src_ref, w_ref, b_ref, g_ref, be_ref, dst_ref):
        def body(t, s):
            h = bn(mm(src_ref[pl.ds(t * TB, TB), :], w_ref[...]) +
                   b_ref[0][None, :], g_ref, be_ref)
            dst_ref[pl.ds(t * TB, TB), :] = h
            return s + jnp.sum(h, axis=0, keepdims=True)
        s = lax.fori_loop(0, T, body, jnp.zeros((1, w_ref.shape[1]),
                                                jnp.float32))
        return s / B

    def col_std(src_ref, mu):
        def body(t, s):
            d = src_ref[pl.ds(t * TB, TB), :] - mu
            return s + jnp.sum(d * d, axis=0, keepdims=True)
        s = lax.fori_loop(0, T, body, jnp.zeros(mu.shape, jnp.float32))
        return jnp.sqrt(s / (B - 1))

    def dice(h, mu, sd, a_ref):
        p = jax.nn.sigmoid((h - mu) / sd)
        return h * p + a_ref[0, 0] * h * (1.0 - p)

    mu1 = layer(x_ref, w1_ref, b1_ref, g1_ref, be1_ref, h1s)
    sd1 = col_std(h1s, mu1)

    def body2(t, s):
        h = dice(h1s[pl.ds(t * TB, TB), :], mu1, sd1, a1_ref)
        h2 = bn(mm(h, w2_ref[...]) + b2_ref[0][None, :], g2_ref, be2_ref)
        h2s[pl.ds(t * TB, TB), :] = h2
        return s + jnp.sum(h2, axis=0, keepdims=True)
    mu2 = lax.fori_loop(0, T, body2,
                        jnp.zeros((1, w2_ref.shape[1]), jnp.float32)) / B
    sd2 = col_std(h2s, mu2)

    def body3(t, _):
        h = dice(h2s[pl.ds(t * TB, TB), :], mu2, sd2, a2_ref)
        o = mm(h, w3_ref[...]) + b3_ref[0][None, :]
        o_ref[pl.ds(t * TB, TB), :] = jax.nn.sigmoid(o)
        return 0
    lax.fori_loop(0, T, body3, 0)


def _tc_mlp(pooled_flat, W1, b1, g1, be1, a1, W2, b2, g2, be2, a2, W3, b3):
    B = pooled_flat.shape[0]
    return pl.pallas_call(
        _mlp_body,
        out_shape=jax.ShapeDtypeStruct((B, 2), jnp.float32),
        scratch_shapes=[
            pltpu.VMEM((B, W1.shape[1]), jnp.float32),
            pltpu.VMEM((B, W2.shape[1]), jnp.float32),
        ],
    )(pooled_flat, W1, b1.reshape(1, -1), g1.reshape(1, -1),
      be1.reshape(1, -1), a1.reshape(1, 1), W2, b2.reshape(1, -1),
      g2.reshape(1, -1), be2.reshape(1, -1), a2.reshape(1, 1), W3,
      b3.reshape(1, -1))


def kernel(context_unixReviewTime, context_itempos, behaviour_itemId,
           behaviour_itemCat, candidate_itemId, candidate_itemCat,
           emb_unixReviewTime, emb_itempos, emb_itemId, emb_itemCat,
           W1, b1, g1, be1, alpha1, W2, b2, g2, be2, alpha2, W3, b3):
    pooled = _sc_pooled_gather(
        context_unixReviewTime, context_itempos, behaviour_itemId,
        behaviour_itemCat, candidate_itemId, candidate_itemCat,
        emb_unixReviewTime, emb_itempos, emb_itemId, emb_itemCat)
    D = emb_itemId.shape[1]
    B = pooled.shape[1]
    perm = (2, 3, 0, 1, 4, 5)
    W1r = jnp.concatenate([W1[f * D:(f + 1) * D] for f in perm], axis=0)
    x = pooled.transpose(1, 0, 2).reshape(B, 6 * D)
    return _tc_mlp(x, W1r, b1, g1, be1, alpha1,
                   W2, b2, g2, be2, alpha2, W3, b3)


# speedup vs baseline: 7.5814x; 1.0386x over previous
import functools

import jax
import jax.numpy as jnp
from jax import lax
from jax.experimental import pallas as pl
from jax.experimental.pallas import tpu as pltpu
from jax.experimental.pallas import tpu_sc as plsc

NC, NS, LANES = 2, 16, 16
NW = NC * NS
BN_EPS = 1e-5
NBUF = 4


def _sc_pooled_gather(idx_urt, idx_pos, idx_bid, idx_bcat, cand_id, cand_cat,
                      t_urt, t_pos, t_id, t_cat):
    L, B = idx_urt.shape
    D = t_id.shape[1]
    bpw = B // NW
    bsc = B // NC
    ngrp = L // NBUF
    mesh = plsc.VectorSubcoreMesh(core_axis_name="c", subcore_axis_name="s",
                                  num_cores=NC, num_subcores=NS)

    @functools.partial(
        pl.kernel,
        out_type=jax.ShapeDtypeStruct((6, B, D), jnp.float32),
        mesh=mesh,
        scratch_types=[
            pltpu.VMEM((2, L, bpw), jnp.int32),
            [pltpu.VMEM((bpw, D), jnp.float32) for _ in range(NBUF)],
            pltpu.VMEM((6, bpw), jnp.int32),
            pltpu.VMEM((bpw, D), jnp.float32),
            pltpu.VMEM_SHARED((6 * (B // NC), D), jnp.float32),
            [pltpu.SemaphoreType.DMA for _ in range(NBUF)],
            [pltpu.SemaphoreType.DMA for _ in range(NBUF)],
            pltpu.SemaphoreType.DMA,
        ],
        compiler_params=pltpu.CompilerParams(use_tc_tiling_on_sc=False),
    )
    def k(urt_h, pos_h, bid_h, bcat_h, cid_h, ccat_h,
          turt_h, tpos_h, tid_h, tcat_h, out_h,
          idxv, stg, dstv, zbuf, acc, gsem, ssem, isem):
        c = lax.axis_index("c")
        s = lax.axis_index("s")
        lb = s * bpw
        gb = c * bsc + lb
        lanes = lax.iota(jnp.int32, LANES)
        zeros16 = jnp.zeros((LANES,), jnp.float32)

        for f in range(6):
            for g in range(bpw // LANES):
                dstv[f, pl.ds(g * LANES, LANES)] = (
                    f * bsc + lb + g * LANES + lanes)
        for r in range(bpw):
            for g in range(D // LANES):
                zbuf[r, pl.ds(g * LANES, LANES)] = zeros16
        for f in range(6):
            pltpu.sync_copy(zbuf, acc.at[pl.ds(f * bsc + lb, bpw), :])

        fields = ((urt_h, turt_h, 0), (pos_h, tpos_h, 1),
                  (bid_h, tid_h, 2), (bcat_h, tcat_h, 3))

        pltpu.async_copy(fields[0][0].at[:, pl.ds(gb, bpw)], idxv.at[0], isem)

        for fi, (idx_h, tab_h, f) in enumerate(fields):
            ib = fi % 2
            idxs = idxv.at[ib]
            pltpu.make_async_copy(
                idx_h.at[:, pl.ds(gb, bpw)], idxs, isem).wait()
            if fi + 1 < len(fields):
                pltpu.async_copy(
                    fields[fi + 1][0].at[:, pl.ds(gb, bpw)],
                    idxv.at[1 - ib], isem)
            dsl = dstv.at[f]

            def fire_g(ci, b, tab_h=tab_h, idxs=idxs):
                pltpu.async_copy(tab_h.at[idxs.at[ci]], stg[b], gsem[b])

            def wait_g(b, tab_h=tab_h, idxs=idxs):
                pltpu.make_async_copy(
                    tab_h.at[idxs.at[0]], stg[b], gsem[b]).wait()

            def fire_s(b, dsl=dsl):
                pltpu.async_copy(stg[b], acc.at[dsl], ssem[b], add=True)

            def wait_s(b, dsl=dsl):
                pltpu.make_async_copy(stg[b], acc.at[dsl], ssem[b]).wait()

            for b in range(NBUF):
                fire_g(b, b)

            def body(g, carry, fire_g=fire_g, wait_g=wait_g,
                     fire_s=fire_s, wait_s=wait_s):
                for b in range(NBUF):
                    wait_g(b)
                    fire_s(b)
                    pb = b - 1 if b > 0 else NBUF - 1
                    if b > 0:
                        @pl.when(g < ngrp - 1)
                        def _(pb=pb, g=g):
                            wait_s(pb)
                            fire_g((g + 1) * NBUF + pb, pb)
                @pl.when(g < ngrp - 1)
                def _():
                    wait_s(NBUF - 1)
                    fire_g((g + 1) * NBUF + NBUF - 1, NBUF - 1)
                return carry

            lax.fori_loop(0, ngrp, body, 0)
            for b in range(NBUF):
                wait_s(b)

        cands = ((cid_h, tid_h, 4), (ccat_h, tcat_h, 5))
        for idx_h, tab_h, f in cands:
            pltpu.sync_copy(idx_h.at[0, pl.ds(gb, bpw)], idxv.at[0, 0])
            pltpu.async_copy(tab_h.at[idxv.at[0, 0]], stg[0], gsem[0]).wait()
            pltpu.sync_copy(stg[0], acc.at[dstv.at[f]], add=True)

        for f in range(6):
            pltpu.sync_copy(acc.at[pl.ds(f * bsc + lb, bpw), :],
                            out_h.at[f, pl.ds(gb, bpw), :])

    return k(idx_urt, idx_pos, idx_bid, idx_bcat,
             cand_id.reshape(1, B), cand_cat.reshape(1, B),
             t_urt, t_pos, t_id, t_cat)


def _mlp_body(x_ref, w1_ref, b1_ref, g1_ref, be1_ref, a1_ref,
              w2_ref, b2_ref, g2_ref, be2_ref, a2_ref,
              w3_ref, b3_ref, o_ref, h1s, h2s):
    B = x_ref.shape[0]
    TB = 512
    T = B // TB

    def mm(a, b):
        return lax.dot_general(a, b, (((1,), (0,)), ((), ())),
                               preferred_element_type=jnp.float32,
                               precision=lax.Precision.HIGHEST)

    def bn(h, g_ref, be_ref):
        return (h * (g_ref[0] / jnp.sqrt(1.0 + BN_EPS))[None, :]
                + be_ref[0][None, :])

    def layer(src_ref, w_ref, b_ref, g_ref, be_ref, dst_ref):
        def body(t, s):
            h = bn(mm(src_ref[pl.ds(t * TB, TB), :], w_ref[...]) +
                   b_ref[0][None, :], g_ref, be_ref)
            dst_ref[pl.ds(t * TB, TB), :] = h
            return s + jnp.sum(h, axis=0, keepdims=True)
        s = lax.fori_loop(0, T, body, jnp.zeros((1, w_ref.shape[1]),
                                                jnp.float32))
        return s / B

    def col_std(src_ref, mu):
        def body(t, s):
            d = src_ref[pl.ds(t * TB, TB), :] - mu
            return s + jnp.sum(d * d, axis=0, keepdims=True)
        s = lax.fori_loop(0, T, body, jnp.zeros(mu.shape, jnp.float32))
        return jnp.sqrt(s / (B - 1))

    def dice(h, mu, sd, a_ref):
        p = jax.nn.sigmoid((h - mu) / sd)
        return h * p + a_ref[0, 0] * h * (1.0 - p)

    mu1 = layer(x_ref, w1_ref, b1_ref, g1_ref, be1_ref, h1s)
    sd1 = col_std(h1s, mu1)

    def body2(t, s):
        h = dice(h1s[pl.ds(t * TB, TB), :], mu1, sd1, a1_ref)
        h2 = bn(mm(h, w2_ref[...]) + b2_ref[0][None, :], g2_ref, be2_ref)
        h2s[pl.ds(t * TB, TB), :] = h2
        return s + jnp.sum(h2, axis=0, keepdims=True)
    mu2 = lax.fori_loop(0, T, body2,
                        jnp.zeros((1, w2_ref.shape[1]), jnp.float32)) / B
    sd2 = col_std(h2s, mu2)

    def body3(t, _):
        h = dice(h2s[pl.ds(t * TB, TB), :], mu2, sd2, a2_ref)
        o = mm(h, w3_ref[...]) + b3_ref[0][None, :]
        o_ref[pl.ds(t * TB, TB), :] = jax.nn.sigmoid(o)
        return 0
    lax.fori_loop(0, T, body3, 0)


def _tc_mlp(pooled_flat, W1, b1, g1, be1, a1, W2, b2, g2, be2, a2, W3, b3):
    B = pooled_flat.shape[0]
    return pl.pallas_call(
        _mlp_body,
        out_shape=jax.ShapeDtypeStruct((B, 2), jnp.float32),
        scratch_shapes=[
            pltpu.VMEM((B, W1.shape[1]), jnp.float32),
            pltpu.VMEM((B, W2.shape[1]), jnp.float32),
        ],
    )(pooled_flat, W1, b1.reshape(1, -1), g1.reshape(1, -1),
      be1.reshape(1, -1), a1.reshape(1, 1), W2, b2.reshape(1, -1),
      g2.reshape(1, -1), be2.reshape(1, -1), a2.reshape(1, 1), W3,
      b3.reshape(1, -1))


def kernel(context_unixReviewTime, context_itempos, behaviour_itemId,
           behaviour_itemCat, candidate_itemId, candidate_itemCat,
           emb_unixReviewTime, emb_itempos, emb_itemId, emb_itemCat,
           W1, b1, g1, be1, alpha1, W2, b2, g2, be2, alpha2, W3, b3):
    pooled = _sc_pooled_gather(
        context_unixReviewTime, context_itempos, behaviour_itemId,
        behaviour_itemCat, candidate_itemId, candidate_itemCat,
        emb_unixReviewTime, emb_itempos, emb_itemId, emb_itemCat)
    D = emb_itemId.shape[1]
    B = pooled.shape[1]
    perm = (2, 3, 0, 1, 4, 5)
    W1r = jnp.concatenate([W1[f * D:(f + 1) * D] for f in perm], axis=0)
    x = pooled.transpose(1, 0, 2).reshape(B, 6 * D)
    return _tc_mlp(x, W1r, b1, g1, be1, alpha1,
                   W2, b2, g2, be2, alpha2, W3, b3)


# speedup vs baseline: 10.4424x; 1.3774x over previous
import functools

import jax
import jax.numpy as jnp
from jax import lax
from jax.experimental import pallas as pl
from jax.experimental.pallas import tpu as pltpu
from jax.experimental.pallas import tpu_sc as plsc

NC, NS, LANES = 2, 16, 16
NW = NC * NS
BN_EPS = 1e-5
NBUF = 4


def _sc_pooled_gather(idx_urt, idx_pos, idx_bid, idx_bcat, cand_id, cand_cat,
                      t_urt, t_pos, t_id, t_cat):
    L, B = idx_urt.shape
    D = t_id.shape[1]
    bpw = B // NW
    bsc = B // NC
    ngrp = L // NBUF
    L_URT, L_POS, L_CAT = t_urt.shape[0], t_pos.shape[0], t_cat.shape[0]
    assert B // NW == 128 and L % 8 == 0
    mesh = plsc.VectorSubcoreMesh(core_axis_name="c", subcore_axis_name="s",
                                  num_cores=NC, num_subcores=NS)

    @functools.partial(
        pl.kernel,
        out_type=jax.ShapeDtypeStruct((6, B, D), jnp.float32),
        mesh=mesh,
        scratch_types=[
            pltpu.VMEM((2, L // 8, 8, bpw), jnp.int32),
            [pltpu.VMEM((bpw, D), jnp.float32) for _ in range(NBUF)],
            pltpu.VMEM((6, bpw), jnp.int32),
            pltpu.VMEM((bpw, D), jnp.float32),
            pltpu.VMEM_SHARED((6 * (B // NC), D), jnp.float32),
            pltpu.VMEM_SHARED((L_URT, D), jnp.float32),
            pltpu.VMEM_SHARED((L_POS, D), jnp.float32),
            pltpu.VMEM_SHARED((L_CAT, D), jnp.float32),
            [pltpu.SemaphoreType.DMA for _ in range(NBUF)],
            [pltpu.SemaphoreType.DMA for _ in range(NBUF)],
            pltpu.SemaphoreType.DMA,
        ],
        compiler_params=pltpu.CompilerParams(use_tc_tiling_on_sc=False),
    )
    def k(urt_h, pos_h, bid_h, bcat_h, cid_h, ccat_h,
          turt_h, tpos_h, tid_h, tcat_h, out_h,
          idxv, stg, dstv, zbuf, acc, tsh_urt, tsh_pos, tsh_cat,
          gsem, ssem, isem):
        c = lax.axis_index("c")
        s = lax.axis_index("s")
        lb = s * bpw
        gb = c * bsc + lb
        lanes = lax.iota(jnp.int32, LANES)
        zeros16 = jnp.zeros((LANES,), jnp.float32)

        for f in range(6):
            for g in range(bpw // LANES):
                dstv[f, pl.ds(g * LANES, LANES)] = (
                    f * bsc + lb + g * LANES + lanes)
        for r in range(bpw):
            for g in range(D // LANES):
                zbuf[r, pl.ds(g * LANES, LANES)] = zeros16
        for f in range(6):
            pltpu.sync_copy(zbuf, acc.at[pl.ds(f * bsc + lb, bpw), :])

        @pl.when(s == 0)
        def _():
            pltpu.sync_copy(turt_h, tsh_urt)
        @pl.when(s == 1)
        def _():
            pltpu.sync_copy(tpos_h, tsh_pos)
        @pl.when(s == 2)
        def _():
            pltpu.sync_copy(tcat_h, tsh_cat)
        plsc.subcore_barrier()

        fields = ((urt_h, tsh_urt, 0), (pos_h, tsh_pos, 1),
                  (bid_h, tid_h, 2), (bcat_h, tsh_cat, 3))

        wid = c * NS + s
        pltpu.async_copy(fields[0][0].at[:, wid, :, :], idxv.at[0], isem)

        for fi, (idx_h, tab_h, f) in enumerate(fields):
            ib = fi % 2
            idxs = idxv.at[ib]
            pltpu.make_async_copy(
                idx_h.at[:, wid, :, :], idxs, isem).wait()
            if fi + 1 < len(fields):
                pltpu.async_copy(
                    fields[fi + 1][0].at[:, wid, :, :],
                    idxv.at[1 - ib], isem)
            dsl = dstv.at[f]

            def fire_g(ci, b, tab_h=tab_h, idxs=idxs):
                pltpu.async_copy(
                    tab_h.at[idxs.at[ci // 8, ci % 8]], stg[b], gsem[b])

            def wait_g(b, tab_h=tab_h, idxs=idxs):
                pltpu.make_async_copy(
                    tab_h.at[idxs.at[0, 0]], stg[b], gsem[b]).wait()

            def fire_s(b, dsl=dsl):
                pltpu.async_copy(stg[b], acc.at[dsl], ssem[b], add=True)

            def wait_s(b, dsl=dsl):
                pltpu.make_async_copy(stg[b], acc.at[dsl], ssem[b]).wait()

            for b in range(NBUF):
                fire_g(b, b)

            def body(g, carry, fire_g=fire_g, wait_g=wait_g,
                     fire_s=fire_s, wait_s=wait_s):
                for b in range(NBUF):
                    wait_g(b)
                    fire_s(b)
                    pb = b - 1 if b > 0 else NBUF - 1
                    if b > 0:
                        @pl.when(g < ngrp - 1)
                        def _(pb=pb, g=g):
                            wait_s(pb)
                            fire_g((g + 1) * NBUF + pb, pb)
                @pl.when(g < ngrp - 1)
                def _():
                    wait_s(NBUF - 1)
                    fire_g((g + 1) * NBUF + NBUF - 1, NBUF - 1)
                return carry

            lax.fori_loop(0, ngrp, body, 0)
            for b in range(NBUF):
                wait_s(b)

        cands = ((cid_h, tid_h, 4), (ccat_h, tsh_cat, 5))
        for idx_h, tab_h, f in cands:
            pltpu.sync_copy(idx_h.at[0, pl.ds(gb, bpw)], idxv.at[0, 0, 0])
            pltpu.async_copy(tab_h.at[idxv.at[0, 0, 0]], stg[0], gsem[0]).wait()
            pltpu.sync_copy(stg[0], acc.at[dstv.at[f]], add=True)

        for f in range(6):
            pltpu.sync_copy(acc.at[pl.ds(f * bsc + lb, bpw), :],
                            out_h.at[f, pl.ds(gb, bpw), :])

    def tile_view(a):
        return a.reshape(L // 8, 8, B // 128, 128).transpose(0, 2, 1, 3)

    return k(tile_view(idx_urt), tile_view(idx_pos), tile_view(idx_bid),
             tile_view(idx_bcat),
             cand_id.reshape(1, B), cand_cat.reshape(1, B),
             t_urt, t_pos, t_id, t_cat)


def _mlp_body(x_ref, w1_ref, b1_ref, g1_ref, be1_ref, a1_ref,
              w2_ref, b2_ref, g2_ref, be2_ref, a2_ref,
              w3_ref, b3_ref, o_ref, h1s, h2s):
    B = x_ref.shape[0]
    TB = 512
    T = B // TB

    def mm(a, b):
        return lax.dot_general(a, b, (((1,), (0,)), ((), ())),
                               preferred_element_type=jnp.float32,
                               precision=lax.Precision.HIGHEST)

    def bn(h, g_ref, be_ref):
        return (h * (g_ref[0] / jnp.sqrt(1.0 + BN_EPS))[None, :]
                + be_ref[0][None, :])

    def layer(src_ref, w_ref, b_ref, g_ref, be_ref, dst_ref):
        def body(t, s):
            h = bn(mm(src_ref[pl.ds(t * TB, TB), :], w_ref[...]) +
                   b_ref[0][None, :], g_ref, be_ref)
            dst_ref[pl.ds(t * TB, TB), :] = h
            return s + jnp.sum(h, axis=0, keepdims=True)
        s = lax.fori_loop(0, T, body, jnp.zeros((1, w_ref.shape[1]),
                                                jnp.float32))
        return s / B

    def col_std(src_ref, mu):
        def body(t, s):
            d = src_ref[pl.ds(t * TB, TB), :] - mu
            return s + jnp.sum(d * d, axis=0, keepdims=True)
        s = lax.fori_loop(0, T, body, jnp.zeros(mu.shape, jnp.float32))
        return jnp.sqrt(s / (B - 1))

    def dice(h, mu, sd, a_ref):
        p = jax.nn.sigmoid((h - mu) / sd)
        return h * p + a_ref[0, 0] * h * (1.0 - p)

    mu1 = layer(x_ref, w1_ref, b1_ref, g1_ref, be1_ref, h1s)
    sd1 = col_std(h1s, mu1)

    def body2(t, s):
        h = dice(h1s[pl.ds(t * TB, TB), :], mu1, sd1, a1_ref)
        h2 = bn(mm(h, w2_ref[...]) + b2_ref[0][None, :], g2_ref, be2_ref)
        h2s[pl.ds(t * TB, TB), :] = h2
        return s + jnp.sum(h2, axis=0, keepdims=True)
    mu2 = lax.fori_loop(0, T, body2,
                        jnp.zeros((1, w2_ref.shape[1]), jnp.float32)) / B
    sd2 = col_std(h2s, mu2)

    def body3(t, _):
        h = dice(h2s[pl.ds(t * TB, TB), :], mu2, sd2, a2_ref)
        o = mm(h, w3_ref[...]) + b3_ref[0][None, :]
        o_ref[pl.ds(t * TB, TB), :] = jax.nn.sigmoid(o)
        return 0
    lax.fori_loop(0, T, body3, 0)


def _tc_mlp(pooled_flat, W1, b1, g1, be1, a1, W2, b2, g2, be2, a2, W3, b3):
    B = pooled_flat.shape[0]
    return pl.pallas_call(
        _mlp_body,
        out_shape=jax.ShapeDtypeStruct((B, 2), jnp.float32),
        scratch_shapes=[
            pltpu.VMEM((B, W1.shape[1]), jnp.float32),
            pltpu.VMEM((B, W2.shape[1]), jnp.float32),
        ],
    )(pooled_flat, W1, b1.reshape(1, -1), g1.reshape(1, -1),
      be1.reshape(1, -1), a1.reshape(1, 1), W2, b2.reshape(1, -1),
      g2.reshape(1, -1), be2.reshape(1, -1), a2.reshape(1, 1), W3,
      b3.reshape(1, -1))


def kernel(context_unixReviewTime, context_itempos, behaviour_itemId,
           behaviour_itemCat, candidate_itemId, candidate_itemCat,
           emb_unixReviewTime, emb_itempos, emb_itemId, emb_itemCat,
           W1, b1, g1, be1, alpha1, W2, b2, g2, be2, alpha2, W3, b3):
    pooled = _sc_pooled_gather(
        context_unixReviewTime, context_itempos, behaviour_itemId,
        behaviour_itemCat, candidate_itemId, candidate_itemCat,
        emb_unixReviewTime, emb_itempos, emb_itemId, emb_itemCat)
    D = emb_itemId.shape[1]
    B = pooled.shape[1]
    perm = (2, 3, 0, 1, 4, 5)
    W1r = jnp.concatenate([W1[f * D:(f + 1) * D] for f in perm], axis=0)
    x = pooled.transpose(1, 0, 2).reshape(B, 6 * D)
    return _tc_mlp(x, W1r, b1, g1, be1, alpha1,
                   W2, b2, g2, be2, alpha2, W3, b3)


# speedup vs baseline: 11.3557x; 1.0875x over previous
import functools

import jax
import jax.numpy as jnp
from jax import lax
from jax.experimental import pallas as pl
from jax.experimental.pallas import tpu as pltpu
from jax.experimental.pallas import tpu_sc as plsc

NC, NS, LANES = 2, 16, 16
NW = NC * NS
BN_EPS = 1e-5
SPL = (128, 72)


def _sc_pooled_gather(idx_urt, idx_pos, idx_bid, idx_bcat, cand_id, cand_cat,
                      t_urt, t_pos, t_id, t_cat):
    L, B = idx_urt.shape
    D = t_id.shape[1]
    bpw = B // NW
    bsc = B // NC
    L_URT, L_POS, L_CAT = t_urt.shape[0], t_pos.shape[0], t_cat.shape[0]
    assert bpw == 128 and L % 8 == 0 and sum(SPL) == L
    mesh = plsc.VectorSubcoreMesh(core_axis_name="c", subcore_axis_name="s",
                                  num_cores=NC, num_subcores=NS)

    @functools.partial(
        pl.kernel,
        out_type=jax.ShapeDtypeStruct((6, B, D), jnp.float32),
        mesh=mesh,
        scratch_types=[
            pltpu.VMEM((2, L // 8, 8, bpw), jnp.int32),
            pltpu.VMEM((bpw, L), jnp.int32),
            pltpu.VMEM((L, D), jnp.float32),
            pltpu.VMEM((L, D), jnp.float32),
            pltpu.VMEM((bpw, D), jnp.float32),
            pltpu.VMEM_SHARED((L_URT, D), jnp.float32),
            pltpu.VMEM_SHARED((L_POS, D), jnp.float32),
            pltpu.VMEM_SHARED((L_CAT, D), jnp.float32),
            pltpu.SemaphoreType.DMA,
            pltpu.SemaphoreType.DMA,
            pltpu.SemaphoreType.DMA,
        ],
        compiler_params=pltpu.CompilerParams(use_tc_tiling_on_sc=False,
                                             needs_layout_passes=False),
    )
    def k(urt_h, pos_h, bid_h, bcat_h, cid_h, ccat_h,
          turt_h, tpos_h, tid_h, tcat_h, out_h,
          idxv, idxt, stga, stgb, obuf, tsh_urt, tsh_pos, tsh_cat,
          sema, semb, isem):
        c = lax.axis_index("c")
        s = lax.axis_index("s")
        wid = c * NS + s
        gb = c * bsc + s * bpw
        lanes = lax.iota(jnp.int32, LANES)

        @pl.when(s == 0)
        def _():
            pltpu.sync_copy(turt_h, tsh_urt)
        @pl.when(s == 1)
        def _():
            pltpu.sync_copy(tpos_h, tsh_pos)
        @pl.when(s == 2)
        def _():
            pltpu.sync_copy(tcat_h, tsh_cat)
        plsc.subcore_barrier()

        fields = ((urt_h, tsh_urt, 0), (pos_h, tsh_pos, 1),
                  (bid_h, tid_h, 2), (bcat_h, tsh_cat, 3))

        pltpu.async_copy(fields[0][0].at[:, wid, :, :], idxv.at[0], isem)

        for fi, (idx_h, tab_h, f) in enumerate(fields):
            ib = fi % 2
            idxs = idxv.at[ib]
            pltpu.make_async_copy(
                idx_h.at[:, wid, :, :], idxs, isem).wait()
            if fi + 1 < len(fields):
                pltpu.async_copy(
                    fields[fi + 1][0].at[:, wid, :, :],
                    idxv.at[1 - ib], isem)

            def tbody(l, carry, idxs=idxs):
                ti = l // 8
                r = l % 8
                col = jnp.full((LANES,), l, jnp.int32)
                for g in range(bpw // LANES):
                    v = idxs[ti, r, pl.ds(g * LANES, LANES)]
                    plsc.store_scatter(idxt, [g * LANES + lanes, col], v)
                return carry
            lax.fori_loop(0, L, tbody, 0)

            def fire(b, stg, sem, tab_h=tab_h):
                o = 0
                for n in SPL:
                    pltpu.async_copy(
                        tab_h.at[idxt.at[b, pl.ds(o, n)]],
                        stg.at[pl.ds(o, n)], sem)
                    o += n

            def wait(stg, sem, tab_h=tab_h):
                o = 0
                for n in SPL:
                    pltpu.make_async_copy(
                        tab_h.at[idxt.at[0, pl.ds(o, n)]],
                        stg.at[pl.ds(o, n)], sem).wait()
                    o += n

            def reduce(b, stg):
                def rbody(i, carry):
                    a0, a1 = carry
                    for u in range(8):
                        a0 = a0 + stg[i * 8 + u, pl.ds(0, LANES)]
                        a1 = a1 + stg[i * 8 + u, pl.ds(LANES, LANES)]
                    return a0, a1
                z = jnp.zeros((LANES,), jnp.float32)
                a0, a1 = lax.fori_loop(0, L // 8, rbody, (z, z))
                obuf[b, pl.ds(0, LANES)] = a0
                obuf[b, pl.ds(LANES, LANES)] = a1

            fire(0, stga, sema)
            fire(1, stgb, semb)

            def body(p, carry, fire=fire, wait=wait, reduce=reduce):
                b0 = 2 * p
                wait(stga, sema)
                reduce(b0, stga)
                @pl.when(p < bpw // 2 - 1)
                def _():
                    fire(b0 + 2, stga, sema)
                wait(stgb, semb)
                reduce(b0 + 1, stgb)
                @pl.when(p < bpw // 2 - 1)
                def _():
                    fire(b0 + 3, stgb, semb)
                return carry

            lax.fori_loop(0, bpw // 2, body, 0)
            pltpu.sync_copy(obuf, out_h.at[f, pl.ds(gb, bpw), :])

        cands = ((cid_h, tid_h, 4), (ccat_h, tsh_cat, 5))
        for idx_h, tab_h, f in cands:
            pltpu.sync_copy(idx_h.at[0, pl.ds(gb, bpw)], idxt.at[0, pl.ds(0, bpw)])
            pltpu.async_copy(
                tab_h.at[idxt.at[0, pl.ds(0, bpw)]], obuf, sema).wait()
            pltpu.sync_copy(obuf, out_h.at[f, pl.ds(gb, bpw), :])

    def tile_view(a):
        return a.reshape(L // 8, 8, B // 128, 128).transpose(0, 2, 1, 3)

    return k(tile_view(idx_urt), tile_view(idx_pos), tile_view(idx_bid),
             tile_view(idx_bcat),
             cand_id.reshape(1, B), cand_cat.reshape(1, B),
             t_urt, t_pos, t_id, t_cat)


def _mlp_body(x_ref, w1_ref, b1_ref, g1_ref, be1_ref, a1_ref,
              w2_ref, b2_ref, g2_ref, be2_ref, a2_ref,
              w3_ref, b3_ref, o_ref, h1s, h2s):
    B = x_ref.shape[0]
    TB = 512
    T = B // TB

    def mm(a, b):
        return lax.dot_general(a, b, (((1,), (0,)), ((), ())),
                               preferred_element_type=jnp.float32,
                               precision=lax.Precision.HIGHEST)

    def bn(h, g_ref, be_ref):
        return (h * (g_ref[0] / jnp.sqrt(1.0 + BN_EPS))[None, :]
                + be_ref[0][None, :])

    def layer(src_ref, w_ref, b_ref, g_ref, be_ref, dst_ref):
        def body(t, s):
            h = bn(mm(src_ref[pl.ds(t * TB, TB), :], w_ref[...]) +
                   b_ref[0][None, :], g_ref, be_ref)
            dst_ref[pl.ds(t * TB, TB), :] = h
            return s + jnp.sum(h, axis=0, keepdims=True)
        s = lax.fori_loop(0, T, body, jnp.zeros((1, w_ref.shape[1]),
                                                jnp.float32))
        return s / B

    def col_std(src_ref, mu):
        def body(t, s):
            d = src_ref[pl.ds(t * TB, TB), :] - mu
            return s + jnp.sum(d * d, axis=0, keepdims=True)
        s = lax.fori_loop(0, T, body, jnp.zeros(mu.shape, jnp.float32))
        return jnp.sqrt(s / (B - 1))

    def dice(h, mu, sd, a_ref):
        p = jax.nn.sigmoid((h - mu) / sd)
        return h * p + a_ref[0, 0] * h * (1.0 - p)

    mu1 = layer(x_ref, w1_ref, b1_ref, g1_ref, be1_ref, h1s)
    sd1 = col_std(h1s, mu1)

    def body2(t, s):
        h = dice(h1s[pl.ds(t * TB, TB), :], mu1, sd1, a1_ref)
        h2 = bn(mm(h, w2_ref[...]) + b2_ref[0][None, :], g2_ref, be2_ref)
        h2s[pl.ds(t * TB, TB), :] = h2
        return s + jnp.sum(h2, axis=0, keepdims=True)
    mu2 = lax.fori_loop(0, T, body2,
                        jnp.zeros((1, w2_ref.shape[1]), jnp.float32)) / B
    sd2 = col_std(h2s, mu2)

    def body3(t, _):
        h = dice(h2s[pl.ds(t * TB, TB), :], mu2, sd2, a2_ref)
        o = mm(h, w3_ref[...]) + b3_ref[0][None, :]
        o_ref[pl.ds(t * TB, TB), :] = jax.nn.sigmoid(o)
        return 0
    lax.fori_loop(0, T, body3, 0)


def _tc_mlp(pooled_flat, W1, b1, g1, be1, a1, W2, b2, g2, be2, a2, W3, b3):
    B = pooled_flat.shape[0]
    return pl.pallas_call(
        _mlp_body,
        out_shape=jax.ShapeDtypeStruct((B, 2), jnp.float32),
        scratch_shapes=[
            pltpu.VMEM((B, W1.shape[1]), jnp.float32),
            pltpu.VMEM((B, W2.shape[1]), jnp.float32),
        ],
    )(pooled_flat, W1, b1.reshape(1, -1), g1.reshape(1, -1),
      be1.reshape(1, -1), a1.reshape(1, 1), W2, b2.reshape(1, -1),
      g2.reshape(1, -1), be2.reshape(1, -1), a2.reshape(1, 1), W3,
      b3.reshape(1, -1))


def kernel(context_unixReviewTime, context_itempos, behaviour_itemId,
           behaviour_itemCat, candidate_itemId, candidate_itemCat,
           emb_unixReviewTime, emb_itempos, emb_itemId, emb_itemCat,
           W1, b1, g1, be1, alpha1, W2, b2, g2, be2, alpha2, W3, b3):
    pooled = _sc_pooled_gather(
        context_unixReviewTime, context_itempos, behaviour_itemId,
        behaviour_itemCat, candidate_itemId, candidate_itemCat,
        emb_unixReviewTime, emb_itempos, emb_itemId, emb_itemCat)
    D = emb_itemId.shape[1]
    B = pooled.shape[1]
    perm = (2, 3, 0, 1, 4, 5)
    W1r = jnp.concatenate([W1[f * D:(f + 1) * D] for f in perm], axis=0)
    x = pooled.transpose(1, 0, 2).reshape(B, 6 * D)
    return _tc_mlp(x, W1r, b1, g1, be1, alpha1,
                   W2, b2, g2, be2, alpha2, W3, b3)


# speedup vs baseline: 15.1400x; 1.3332x over previous
import functools

import jax
import jax.numpy as jnp
from jax import lax
from jax.experimental import pallas as pl
from jax.experimental.pallas import tpu as pltpu
from jax.experimental.pallas import tpu_sc as plsc

NC, NS, LANES = 2, 16, 16
NW = NC * NS
BN_EPS = 1e-5
SPL = (128, 72)


def _rho(v):
    return ((v >> 12) << 12) + ((v & 1023) << 2) + ((v >> 10) & 3)


def _sc_pooled_gather(idx_urt, idx_pos, idx_bid, idx_bcat, cand_id, cand_cat,
                      t_urt, t_pos, t_id, t_cat):
    L, B = idx_urt.shape
    D = t_id.shape[1]
    bpw = B // NW
    bsc = B // NC
    L_URT, L_POS, L_CAT = t_urt.shape[0], t_pos.shape[0], t_cat.shape[0]
    assert bpw == 128 and L % 8 == 0 and sum(SPL) == L
    mesh = plsc.VectorSubcoreMesh(core_axis_name="c", subcore_axis_name="s",
                                  num_cores=NC, num_subcores=NS)

    @functools.partial(
        pl.kernel,
        out_type=jax.ShapeDtypeStruct((6, B, D), jnp.float32),
        mesh=mesh,
        scratch_types=[
            pltpu.VMEM((2, L // 8, 8, bpw), jnp.int32),
            pltpu.VMEM((bpw, L), jnp.int32),
            pltpu.VMEM((L, D), jnp.float32),
            pltpu.VMEM((L, D), jnp.float32),
            pltpu.VMEM((bpw, D), jnp.float32),
            pltpu.VMEM_SHARED((L_URT, D), jnp.float32),
            pltpu.VMEM_SHARED((L_POS, D), jnp.float32),
            pltpu.VMEM_SHARED((L_CAT, D), jnp.float32),
            pltpu.SemaphoreType.DMA,
            pltpu.SemaphoreType.DMA,
            pltpu.SemaphoreType.DMA,
        ],
        compiler_params=pltpu.CompilerParams(use_tc_tiling_on_sc=False,
                                             needs_layout_passes=False),
    )
    def k(urt_h, pos_h, bid_h, bcat_h, cid_h, ccat_h,
          turt_h, tpos_h, tid_h, tcat_h, out_h,
          idxv, idxt, stga, stgb, obuf, tsh_urt, tsh_pos, tsh_cat,
          sema, semb, isem):
        c = lax.axis_index("c")
        s = lax.axis_index("s")
        wid = c * NS + s
        gb = c * bsc + s * bpw
        lanes = lax.iota(jnp.int32, LANES)

        @pl.when(s == 0)
        def _():
            pltpu.sync_copy(turt_h, tsh_urt)
        @pl.when(s == 1)
        def _():
            pltpu.sync_copy(tpos_h, tsh_pos)
        @pl.when(s == 2)
        def _():
            pltpu.sync_copy(tcat_h, tsh_cat)
        plsc.subcore_barrier()

        fields = ((urt_h, tsh_urt, 0), (pos_h, tsh_pos, 1),
                  (bid_h, tid_h, 2), (bcat_h, tsh_cat, 3))

        pltpu.async_copy(fields[0][0].at[:, wid, :, :], idxv.at[0], isem)

        for fi, (idx_h, tab_h, f) in enumerate(fields):
            ib = fi % 2
            idxs = idxv.at[ib]
            pltpu.make_async_copy(
                idx_h.at[:, wid, :, :], idxs, isem).wait()
            if fi + 1 < len(fields):
                pltpu.async_copy(
                    fields[fi + 1][0].at[:, wid, :, :],
                    idxv.at[1 - ib], isem)

            def tbody(l, carry, idxs=idxs):
                ti = l // 8
                r = l % 8
                col = jnp.full((LANES,), l, jnp.int32)
                for g in range(bpw // LANES):
                    v = idxs[ti, r, pl.ds(g * LANES, LANES)]
                    if fi == 2:
                        v = _rho(v)
                    plsc.store_scatter(idxt, [g * LANES + lanes, col], v)
                return carry
            lax.fori_loop(0, L, tbody, 0)

            def fire(b, stg, sem, tab_h=tab_h):
                o = 0
                for n in SPL:
                    pltpu.async_copy(
                        tab_h.at[idxt.at[b, pl.ds(o, n)]],
                        stg.at[pl.ds(o, n)], sem)
                    o += n

            def wait(stg, sem, tab_h=tab_h):
                o = 0
                for n in SPL:
                    pltpu.make_async_copy(
                        tab_h.at[idxt.at[0, pl.ds(o, n)]],
                        stg.at[pl.ds(o, n)], sem).wait()
                    o += n

            def reduce(b, stg):
                def rbody(i, carry):
                    a0, a1 = carry
                    for u in range(8):
                        a0 = a0 + stg[i * 8 + u, pl.ds(0, LANES)]
                        a1 = a1 + stg[i * 8 + u, pl.ds(LANES, LANES)]
                    return a0, a1
                z = jnp.zeros((LANES,), jnp.float32)
                a0, a1 = lax.fori_loop(0, L // 8, rbody, (z, z))
                obuf[b, pl.ds(0, LANES)] = a0
                obuf[b, pl.ds(LANES, LANES)] = a1

            fire(0, stga, sema)
            fire(1, stgb, semb)

            def body(p, carry, fire=fire, wait=wait, reduce=reduce):
                b0 = 2 * p
                wait(stga, sema)
                reduce(b0, stga)
                @pl.when(p < bpw // 2 - 1)
                def _():
                    fire(b0 + 2, stga, sema)
                wait(stgb, semb)
                reduce(b0 + 1, stgb)
                @pl.when(p < bpw // 2 - 1)
                def _():
                    fire(b0 + 3, stgb, semb)
                return carry

            lax.fori_loop(0, bpw // 2, body, 0)
            pltpu.sync_copy(obuf, out_h.at[f, pl.ds(gb, bpw), :])

        cands = ((cid_h, tid_h, 4, True), (ccat_h, tsh_cat, 5, False))
        for idx_h, tab_h, f, remap in cands:
            pltpu.sync_copy(idx_h.at[0, pl.ds(gb, bpw)], idxt.at[0, pl.ds(0, bpw)])
            if remap:
                for g in range(bpw // LANES):
                    v = idxt[0, pl.ds(g * LANES, LANES)]
                    idxt[0, pl.ds(g * LANES, LANES)] = _rho(v)
            pltpu.async_copy(
                tab_h.at[idxt.at[0, pl.ds(0, bpw)]], obuf, sema).wait()
            pltpu.sync_copy(obuf, out_h.at[f, pl.ds(gb, bpw), :])

    def tile_view(a):
        return a.reshape(L // 8, 8, B // 128, 128).transpose(0, 2, 1, 3)

    return k(tile_view(idx_urt), tile_view(idx_pos), tile_view(idx_bid),
             tile_view(idx_bcat),
             cand_id.reshape(1, B), cand_cat.reshape(1, B),
             t_urt, t_pos, t_id, t_cat)


_RELAYOUT_CC = 4096


def _relayout_body(x_ref, o_ref):
    x = x_ref[...]
    q = _RELAYOUT_CC // 4
    o_ref[...] = jnp.concatenate(
        [x[:, j * q:(j + 1) * q].T for j in range(4)], axis=1)


def _tc_relayout(tT):
    D, R = tT.shape
    n = pl.cdiv(R, _RELAYOUT_CC)
    return pl.pallas_call(
        _relayout_body,
        grid=(n,),
        in_specs=[pl.BlockSpec((D, _RELAYOUT_CC), lambda i: (0, i))],
        out_specs=pl.BlockSpec((_RELAYOUT_CC // 4, 128), lambda i: (i, 0)),
        out_shape=jax.ShapeDtypeStruct((n * _RELAYOUT_CC // 4, 128),
                                       jnp.float32),
    )(tT)


def _mlp_body(x_ref, w1_ref, b1_ref, g1_ref, be1_ref, a1_ref,
              w2_ref, b2_ref, g2_ref, be2_ref, a2_ref,
              w3_ref, b3_ref, o_ref, h1s, h2s):
    B = x_ref.shape[0]
    TB = 512
    T = B // TB

    def mm(a, b):
        return lax.dot_general(a, b, (((1,), (0,)), ((), ())),
                               preferred_element_type=jnp.float32,
                               precision=lax.Precision.HIGHEST)

    def bn(h, g_ref, be_ref):
        return (h * (g_ref[0] / jnp.sqrt(1.0 + BN_EPS))[None, :]
                + be_ref[0][None, :])

    def layer(src_ref, w_ref, b_ref, g_ref, be_ref, dst_ref):
        def body(t, s):
            h = bn(mm(src_ref[pl.ds(t * TB, TB), :], w_ref[...]) +
                   b_ref[0][None, :], g_ref, be_ref)
            dst_ref[pl.ds(t * TB, TB), :] = h
            return s + jnp.sum(h, axis=0, keepdims=True)
        s = lax.fori_loop(0, T, body, jnp.zeros((1, w_ref.shape[1]),
                                                jnp.float32))
        return s / B

    def col_std(src_ref, mu):
        def body(t, s):
            d = src_ref[pl.ds(t * TB, TB), :] - mu
            return s + jnp.sum(d * d, axis=0, keepdims=True)
        s = lax.fori_loop(0, T, body, jnp.zeros(mu.shape, jnp.float32))
        return jnp.sqrt(s / (B - 1))

    def dice(h, mu, sd, a_ref):
        p = jax.nn.sigmoid((h - mu) / sd)
        return h * p + a_ref[0, 0] * h * (1.0 - p)

    mu1 = layer(x_ref, w1_ref, b1_ref, g1_ref, be1_ref, h1s)
    sd1 = col_std(h1s, mu1)

    def body2(t, s):
        h = dice(h1s[pl.ds(t * TB, TB), :], mu1, sd1, a1_ref)
        h2 = bn(mm(h, w2_ref[...]) + b2_ref[0][None, :], g2_ref, be2_ref)
        h2s[pl.ds(t * TB, TB), :] = h2
        return s + jnp.sum(h2, axis=0, keepdims=True)
    mu2 = lax.fori_loop(0, T, body2,
                        jnp.zeros((1, w2_ref.shape[1]), jnp.float32)) / B
    sd2 = col_std(h2s, mu2)

    def body3(t, _):
        h = dice(h2s[pl.ds(t * TB, TB), :], mu2, sd2, a2_ref)
        o = mm(h, w3_ref[...]) + b3_ref[0][None, :]
        o_ref[pl.ds(t * TB, TB), :] = jax.nn.sigmoid(o)
        return 0
    lax.fori_loop(0, T, body3, 0)


def _tc_mlp(pooled_flat, W1, b1, g1, be1, a1, W2, b2, g2, be2, a2, W3, b3):
    B = pooled_flat.shape[0]
    return pl.pallas_call(
        _mlp_body,
        out_shape=jax.ShapeDtypeStruct((B, 2), jnp.float32),
        scratch_shapes=[
            pltpu.VMEM((B, W1.shape[1]), jnp.float32),
            pltpu.VMEM((B, W2.shape[1]), jnp.float32),
        ],
    )(pooled_flat, W1, b1.reshape(1, -1), g1.reshape(1, -1),
      be1.reshape(1, -1), a1.reshape(1, 1), W2, b2.reshape(1, -1),
      g2.reshape(1, -1), be2.reshape(1, -1), a2.reshape(1, 1), W3,
      b3.reshape(1, -1))


def kernel(context_unixReviewTime, context_itempos, behaviour_itemId,
           behaviour_itemCat, candidate_itemId, candidate_itemCat,
           emb_unixReviewTime, emb_itempos, emb_itemId, emb_itemCat,
           W1, b1, g1, be1, alpha1, W2, b2, g2, be2, alpha2, W3, b3):
    D0 = emb_itemId.shape[1]
    t_id_lin = _tc_relayout(emb_itemId.T).reshape(-1, D0)
    pooled = _sc_pooled_gather(
        context_unixReviewTime, context_itempos, behaviour_itemId,
        behaviour_itemCat, candidate_itemId, candidate_itemCat,
        emb_unixReviewTime, emb_itempos, t_id_lin, emb_itemCat)
    D = emb_itemId.shape[1]
    B = pooled.shape[1]
    perm = (2, 3, 0, 1, 4, 5)
    W1r = jnp.concatenate([W1[f * D:(f + 1) * D] for f in perm], axis=0)
    x = pooled.transpose(1, 0, 2).reshape(B, 6 * D)
    return _tc_mlp(x, W1r, b1, g1, be1, alpha1,
                   W2, b2, g2, be2, alpha2, W3, b3)
